# baseline (speedup 1.0000x reference)
import functools

import jax
import jax.numpy as jnp
from jax import lax
from jax.experimental import pallas as pl
from jax.experimental.pallas import tpu as pltpu
from jax.experimental.pallas import tpu_sc as plsc

B = 4
S = 2048
D = 768
L = 16
NV = D // L

_info = plsc.get_sparse_core_info()
NC = _info.num_cores
NS = _info.num_subcores
NW = NC * NS
PPW = S // NW


_GATHER_DNUMS = lax.GatherDimensionNumbers(
    offset_dims=(), collapsed_slice_dims=(0,), start_index_map=(0,))


def _xlane(x, pm):
    return lax.gather(x, pm[:, None], _GATHER_DNUMS, slice_sizes=(1,),
                      mode=lax.GatherScatterMode.PROMISE_IN_BOUNDS)


def _make_kernel():
    mesh = plsc.VectorSubcoreMesh(core_axis_name="c", subcore_axis_name="s")

    @functools.partial(
        pl.kernel,
        mesh=mesh,
        out_type=jax.ShapeDtypeStruct((B, S, D), jnp.float32),
        scratch_types=[
            pltpu.VMEM((B, PPW), jnp.int32),
            pltpu.VMEM((PPW, D), jnp.float32),
            pltpu.VMEM((PPW, D), jnp.float32),
            pltpu.VMEM((D,), jnp.float32),
            pltpu.VMEM((D,), jnp.float32),
            pltpu.SemaphoreType.DMA,
        ],
    )
    def emb_ln(x_hbm, w_hbm, p_hbm, g_hbm, be_hbm, out_hbm,
               idx_v, p_v, rows_v, g_v, be_v, sem):
        wid = lax.axis_index("s") * NC + lax.axis_index("c")
        pos0 = wid * PPW

        pltpu.sync_copy(g_hbm, g_v)
        pltpu.sync_copy(be_hbm, be_v)
        pltpu.sync_copy(p_hbm.at[pl.ds(pos0, PPW), :], p_v)
        for b in range(B):
            pltpu.sync_copy(x_hbm.at[b, pl.ds(pos0, PPW)], idx_v.at[b])

        lane = lax.iota(jnp.int32, L)
        perms = [lane ^ k for k in (8, 4, 2, 1)]

        def row_body(r, carry):
            acc = jnp.zeros((L,), jnp.float32)
            acc2 = jnp.zeros((L,), jnp.float32)
            for j in range(NV):
                sl = pl.ds(j * L, L)
                v = rows_v[r, sl] + p_v[r, sl]
                rows_v[r, sl] = v
                acc = acc + v
                acc2 = acc2 + v * v
            for pm in perms:
                acc = acc + _xlane(acc, pm)
                acc2 = acc2 + _xlane(acc2, pm)
            meanv = acc * (1.0 / D)
            varv = acc2 * (1.0 / D) - meanv * meanv
            xv = varv + 1e-5
            iv = lax.bitcast_convert_type(xv, jnp.int32)
            iv = jnp.int32(0x5F3759DF) - lax.shift_right_logical(iv, 1)
            y = lax.bitcast_convert_type(iv, jnp.float32)
            for _ in range(3):
                y = y * (1.5 - 0.5 * xv * y * y)
            for j in range(NV):
                sl = pl.ds(j * L, L)
                t = (rows_v[r, sl] - meanv) * y
                rows_v[r, sl] = t * g_v[sl] + be_v[sl]
            return carry

        for b in range(B):
            pltpu.async_copy(w_hbm.at[idx_v.at[b]], rows_v, sem).wait()
            lax.fori_loop(0, PPW, row_body, 0)
            pltpu.sync_copy(rows_v, out_hbm.at[b, pl.ds(pos0, PPW), :])

    return emb_ln


_emb_ln = _make_kernel()


@jax.jit
def kernel(x, W, P, gamma, beta):
    return _emb_ln(x.astype(jnp.int32), W, P, gamma, beta)
